# baseline (speedup 1.0000x reference)
import functools

import jax
import jax.numpy as jnp
from jax import lax
from jax.experimental import pallas as pl
from jax.experimental.pallas import tpu as pltpu

_N = 32768
_D = 128
_K = 8
_INT_MAX = 2**31 - 1


def _colsum(xz_ref, num_windows):
    rows_w = _N // num_windows

    def it(i, accs):
        out = []
        for w, acc in enumerate(accs):
            for k in range(4):
                acc = acc + xz_ref[pl.ds(w * rows_w + i * 32 + k * 8, 8), :]
            out.append(acc)
        return tuple(out)

    init = tuple(jnp.zeros((8, _D), jnp.float32) for _ in range(num_windows))
    accs = lax.fori_loop(0, rows_w // 32, it, init)
    return [_tree(acc) for acc in accs]


def _tree(acc):
    v = acc[0:4] + acc[4:8]
    v = v[0:2] + v[2:4]
    return v[0:1] + v[1:2]


def _window_chain(xz_ref, base, rows_w):
    def it(i, acc):
        a = acc
        for k in range(4):
            a = a + xz_ref[pl.ds(base + i * 32 + k * 8, 8), :]
        return a

    return lax.fori_loop(0, rows_w // 32, it, jnp.zeros((8, _D), jnp.float32))


def _matvec(mean_row, x_ref):
    return lax.dot_general(
        mean_row,
        x_ref[...],
        (((1,), (1,)), ((), ())),
        preferred_element_type=jnp.float32,
    )


def _body(x_hbm, out_ref, x_ref, xz_ref, taken_ref, sel_ref, wsums_ref,
          in_sems, cp_sems):
    nch = 8
    ch = _N // nch
    in_copies = []
    for c in range(nch):
        cp = pltpu.make_async_copy(
            x_hbm.at[pl.ds(c * ch, ch), :],
            x_ref.at[pl.ds(c * ch, ch), :],
            in_sems.at[c],
        )
        cp.start()
        in_copies.append(cp)
    xz_copies = []
    taken_ref[...] = jnp.zeros((1, _N), jnp.float32)
    iota_row = lax.broadcasted_iota(jnp.int32, (1, _N), 1)
    prev_idx = None
    sel_vals = []
    count = jnp.float32(32768.0)

    for t in range(_K):
        taken = taken_ref[...]
        if t == 0:
            parts = []
            for w in range(4):
                for c in (2 * w, 2 * w + 1):
                    in_copies[c].wait()
                    cp = pltpu.make_async_copy(
                        x_ref.at[pl.ds(c * ch, ch), :],
                        xz_ref.at[pl.ds(c * ch, ch), :],
                        cp_sems.at[c],
                    )
                    cp.start()
                    xz_copies.append(cp)
                parts.append(_tree(_window_chain(x_ref, w * 8192, 8192)))
            colsum = parts[0]
            for p in parts[1:]:
                colsum = colsum + p
        elif t == 1:
            parts = _colsum(xz_ref, 8)
            for w in range(8):
                wsums_ref[w : w + 1, :] = parts[w]
            colsum = parts[0]
            for p in parts[1:]:
                colsum = colsum + p
        else:
            wstar = prev_idx // 4096
            acc = _window_chain(xz_ref, wstar * 4096, 4096)
            wsums_ref[pl.ds(wstar, 1), :] = _tree(acc)
            v = wsums_ref[...]
            colsum = v[0:1]
            for w in range(1, 8):
                colsum = colsum + v[w : w + 1]
        mean_row = colsum / count
        scores = _matvec(mean_row, x_ref)
        z = 1.0 / scores + taken
        m = jnp.max(z)
        cand = jnp.where(z == m, iota_row, jnp.full((1, _N), _INT_MAX, jnp.int32))
        idx = jnp.min(cand)
        sel_ref[t] = idx
        prev_idx = idx
        taken_ref[...] = jnp.where(iota_row == idx, taken - 100000.0, taken)
        is_dup = jnp.bool_(False)
        for v in sel_vals:
            is_dup = jnp.logical_or(is_dup, v == idx)
        count = count - jnp.where(is_dup, 0.0, 1.0).astype(jnp.float32)
        sel_vals.append(idx)
        if t == 0:
            for cp in xz_copies:
                cp.wait()
        xz_ref[pl.ds(idx, 1), :] = jnp.zeros((1, _D), jnp.float32)

    vals = [sel_ref[k] for k in range(_K)]
    for k in range(_K):
        rank = jnp.int32(0)
        for j in range(_K):
            if j < k:
                rank += (vals[j] <= vals[k]).astype(jnp.int32)
            elif j > k:
                rank += (vals[j] < vals[k]).astype(jnp.int32)
        out_ref[pl.ds(rank, 1), :] = x_ref[pl.ds(vals[k], 1), :]


@jax.jit
def kernel(x):
    return pl.pallas_call(
        _body,
        out_shape=jax.ShapeDtypeStruct((_K, _D), jnp.float32),
        in_specs=[pl.BlockSpec(memory_space=pltpu.MemorySpace.HBM)],
        out_specs=pl.BlockSpec(memory_space=pltpu.VMEM),
        scratch_shapes=[
            pltpu.VMEM((_N, _D), jnp.float32),
            pltpu.VMEM((_N, _D), jnp.float32),
            pltpu.VMEM((1, _N), jnp.float32),
            pltpu.SMEM((_K,), jnp.int32),
            pltpu.VMEM((8, _D), jnp.float32),
            pltpu.SemaphoreType.DMA((8,)),
            pltpu.SemaphoreType.DMA((8,)),
        ],
    )(x)


# speedup vs baseline: 1.0172x; 1.0172x over previous
import functools

import jax
import jax.numpy as jnp
from jax import lax
from jax.experimental import pallas as pl
from jax.experimental.pallas import tpu as pltpu

_N = 32768
_D = 128
_K = 8
_INT_MAX = 2**31 - 1


def _colsum(xz_ref, num_windows):
    rows_w = _N // num_windows

    def it(i, accs):
        out = []
        for w, acc in enumerate(accs):
            for k in range(4):
                acc = acc + xz_ref[pl.ds(w * rows_w + i * 32 + k * 8, 8), :]
            out.append(acc)
        return tuple(out)

    init = tuple(jnp.zeros((8, _D), jnp.float32) for _ in range(num_windows))
    accs = lax.fori_loop(0, rows_w // 32, it, init)
    return [_tree(acc) for acc in accs]


def _tree(acc):
    v = acc[0:4] + acc[4:8]
    v = v[0:2] + v[2:4]
    return v[0:1] + v[1:2]


def _window_chain(xz_ref, base, rows_w):
    def it(i, acc):
        a = acc
        for k in range(4):
            a = a + xz_ref[pl.ds(base + i * 32 + k * 8, 8), :]
        return a

    return lax.fori_loop(0, rows_w // 32, it, jnp.zeros((8, _D), jnp.float32))


def _matvec(mean_row, x_ref):
    return lax.dot_general(
        mean_row,
        x_ref[...],
        (((1,), (1,)), ((), ())),
        preferred_element_type=jnp.float32,
    )


def _body(x_ref, out_ref, xz_ref, taken_ref, sel_ref, wsums_ref, copy_sem):
    copy = pltpu.make_async_copy(x_ref, xz_ref, copy_sem)
    copy.start()
    taken_ref[...] = jnp.zeros((1, _N), jnp.float32)
    iota_row = lax.broadcasted_iota(jnp.int32, (1, _N), 1)
    prev_idx = None
    sel_vals = []
    count = jnp.float32(32768.0)

    for t in range(_K):
        taken = taken_ref[...]
        if t == 0:
            parts = _colsum(x_ref, 4)
            colsum = parts[0]
            for p in parts[1:]:
                colsum = colsum + p
        elif t == 1:
            parts = _colsum(xz_ref, 8)
            for w in range(8):
                wsums_ref[w : w + 1, :] = parts[w]
            colsum = parts[0]
            for p in parts[1:]:
                colsum = colsum + p
        else:
            wstar = prev_idx // 4096
            acc = _window_chain(xz_ref, wstar * 4096, 4096)
            wsums_ref[pl.ds(wstar, 1), :] = _tree(acc)
            v = wsums_ref[...]
            colsum = v[0:1]
            for w in range(1, 8):
                colsum = colsum + v[w : w + 1]
        mean_row = colsum / count
        scores = _matvec(mean_row, x_ref)
        z = 1.0 / scores + taken
        m = jnp.max(z)
        cand = jnp.where(z == m, iota_row, jnp.full((1, _N), _INT_MAX, jnp.int32))
        idx = jnp.min(cand)
        sel_ref[t] = idx
        prev_idx = idx
        taken_ref[...] = jnp.where(iota_row == idx, taken - 100000.0, taken)
        is_dup = jnp.bool_(False)
        for v in sel_vals:
            is_dup = jnp.logical_or(is_dup, v == idx)
        count = count - jnp.where(is_dup, 0.0, 1.0).astype(jnp.float32)
        sel_vals.append(idx)
        if t == 0:
            copy.wait()
        xz_ref[pl.ds(idx, 1), :] = jnp.zeros((1, _D), jnp.float32)

    vals = [sel_ref[k] for k in range(_K)]
    for k in range(_K):
        rank = jnp.int32(0)
        for j in range(_K):
            if j < k:
                rank += (vals[j] <= vals[k]).astype(jnp.int32)
            elif j > k:
                rank += (vals[j] < vals[k]).astype(jnp.int32)
        out_ref[pl.ds(rank, 1), :] = x_ref[pl.ds(vals[k], 1), :]


@jax.jit
def kernel(x):
    return pl.pallas_call(
        _body,
        out_shape=jax.ShapeDtypeStruct((_K, _D), jnp.float32),
        in_specs=[pl.BlockSpec(memory_space=pltpu.VMEM)],
        out_specs=pl.BlockSpec(memory_space=pltpu.VMEM),
        scratch_shapes=[
            pltpu.VMEM((_N, _D), jnp.float32),
            pltpu.VMEM((1, _N), jnp.float32),
            pltpu.SMEM((_K,), jnp.int32),
            pltpu.VMEM((8, _D), jnp.float32),
            pltpu.SemaphoreType.DMA,
        ],
    )(x)


# speedup vs baseline: 1.0353x; 1.0178x over previous
import functools

import jax
import jax.numpy as jnp
from jax import lax
from jax.experimental import pallas as pl
from jax.experimental.pallas import tpu as pltpu

_N = 32768
_D = 128
_K = 8
_INT_MAX = 2**31 - 1


def _colsum(xz_ref, num_windows):
    rows_w = _N // num_windows

    def it(i, accs):
        out = []
        for w, acc in enumerate(accs):
            for k in range(4):
                acc = acc + xz_ref[pl.ds(w * rows_w + i * 32 + k * 8, 8), :]
            out.append(acc)
        return tuple(out)

    init = tuple(jnp.zeros((8, _D), jnp.float32) for _ in range(num_windows))
    accs = lax.fori_loop(0, rows_w // 32, it, init)
    return [_tree(acc) for acc in accs]


def _tree(acc):
    v = acc[0:4] + acc[4:8]
    v = v[0:2] + v[2:4]
    return v[0:1] + v[1:2]


def _window_chain(xz_ref, base, rows_w):
    def it(i, acc):
        a = acc
        for k in range(4):
            a = a + xz_ref[pl.ds(base + i * 32 + k * 8, 8), :]
        return a

    return lax.fori_loop(0, rows_w // 32, it, jnp.zeros((8, _D), jnp.float32))


def _matvec(mean_row, x_ref):
    return lax.dot_general(
        mean_row,
        x_ref[...],
        (((1,), (1,)), ((), ())),
        preferred_element_type=jnp.float32,
    )


def _body(x_ref, out_ref, xz_ref, taken_ref, z_ref, sel_ref, wsums_ref,
          copy_sem):
    copy = pltpu.make_async_copy(x_ref, xz_ref, copy_sem)
    copy.start()
    taken_ref[...] = jnp.zeros((1, _N), jnp.float32)
    iota_row = lax.broadcasted_iota(jnp.int32, (1, _N), 1)
    prev_idx = None
    sel_vals = []
    count = jnp.float32(32768.0)

    for t in range(_K):
        taken = taken_ref[...]
        if t == 0:
            parts = _colsum(x_ref, 4)
            colsum = parts[0]
            for p in parts[1:]:
                colsum = colsum + p
        elif t == 1:
            parts = _colsum(xz_ref, 8)
            for w in range(8):
                wsums_ref[w : w + 1, :] = parts[w]
            colsum = parts[0]
            for p in parts[1:]:
                colsum = colsum + p
        else:
            wstar = prev_idx // 4096
            acc = _window_chain(xz_ref, wstar * 4096, 4096)
            wsums_ref[pl.ds(wstar, 1), :] = _tree(acc)
            v = wsums_ref[...]
            colsum = v[0:1]
            for w in range(1, 8):
                colsum = colsum + v[w : w + 1]
        mean_row = colsum / count
        chn = _N // 4
        ms = []
        for c in range(4):
            sc = lax.dot_general(
                mean_row,
                x_ref[pl.ds(c * chn, chn), :],
                (((1,), (1,)), ((), ())),
                preferred_element_type=jnp.float32,
            )
            zc = 1.0 / sc + taken[0:1, c * chn : (c + 1) * chn]
            z_ref[0:1, c * chn : (c + 1) * chn] = zc
            ms.append(jnp.max(zc))
        m = jnp.maximum(jnp.maximum(ms[0], ms[1]), jnp.maximum(ms[2], ms[3]))
        z = z_ref[...]
        cand = jnp.where(z == m, iota_row, jnp.full((1, _N), _INT_MAX, jnp.int32))
        idx = jnp.min(cand)
        sel_ref[t] = idx
        prev_idx = idx
        taken_ref[...] = jnp.where(iota_row == idx, taken - 100000.0, taken)
        is_dup = jnp.bool_(False)
        for v in sel_vals:
            is_dup = jnp.logical_or(is_dup, v == idx)
        count = count - jnp.where(is_dup, 0.0, 1.0).astype(jnp.float32)
        sel_vals.append(idx)
        if t == 0:
            copy.wait()
        xz_ref[pl.ds(idx, 1), :] = jnp.zeros((1, _D), jnp.float32)

    vals = [sel_ref[k] for k in range(_K)]
    for k in range(_K):
        rank = jnp.int32(0)
        for j in range(_K):
            if j < k:
                rank += (vals[j] <= vals[k]).astype(jnp.int32)
            elif j > k:
                rank += (vals[j] < vals[k]).astype(jnp.int32)
        out_ref[pl.ds(rank, 1), :] = x_ref[pl.ds(vals[k], 1), :]


@jax.jit
def kernel(x):
    return pl.pallas_call(
        _body,
        out_shape=jax.ShapeDtypeStruct((_K, _D), jnp.float32),
        in_specs=[pl.BlockSpec(memory_space=pltpu.VMEM)],
        out_specs=pl.BlockSpec(memory_space=pltpu.VMEM),
        scratch_shapes=[
            pltpu.VMEM((_N, _D), jnp.float32),
            pltpu.VMEM((1, _N), jnp.float32),
            pltpu.VMEM((1, _N), jnp.float32),
            pltpu.SMEM((_K,), jnp.int32),
            pltpu.VMEM((8, _D), jnp.float32),
            pltpu.SemaphoreType.DMA,
        ],
    )(x)


# speedup vs baseline: 1.0572x; 1.0211x over previous
import functools

import jax
import jax.numpy as jnp
from jax import lax
from jax.experimental import pallas as pl
from jax.experimental.pallas import tpu as pltpu

_N = 32768
_D = 128
_K = 8
_INT_MAX = 2**31 - 1


def _colsum(xz_ref, num_windows):
    rows_w = _N // num_windows

    def it(i, accs):
        out = []
        for w, acc in enumerate(accs):
            for k in range(4):
                acc = acc + xz_ref[pl.ds(w * rows_w + i * 32 + k * 8, 8), :]
            out.append(acc)
        return tuple(out)

    init = tuple(jnp.zeros((8, _D), jnp.float32) for _ in range(num_windows))
    accs = lax.fori_loop(0, rows_w // 32, it, init)
    return [_tree(acc) for acc in accs]


def _tree(acc):
    v = acc[0:4] + acc[4:8]
    v = v[0:2] + v[2:4]
    return v[0:1] + v[1:2]


def _window_chain(xz_ref, base, rows_w):
    def it(i, acc):
        a = acc
        for k in range(4):
            a = a + xz_ref[pl.ds(base + i * 32 + k * 8, 8), :]
        return a

    return lax.fori_loop(0, rows_w // 32, it, jnp.zeros((8, _D), jnp.float32))


def _matvec(mean_row, x_ref):
    return lax.dot_general(
        mean_row,
        x_ref[...],
        (((1,), (1,)), ((), ())),
        preferred_element_type=jnp.float32,
    )


def _body(x_ref, out_ref, xz_ref, taken_ref, z_ref, sel_ref, wsums_ref,
          copy_sem):
    copy = pltpu.make_async_copy(x_ref, xz_ref, copy_sem)
    copy.start()
    taken_ref[...] = jnp.zeros((1, _N), jnp.float32)
    iota_row = lax.broadcasted_iota(jnp.int32, (1, _N), 1)
    iota_chunk = lax.broadcasted_iota(jnp.int32, (1, _N // 4), 1)
    prev_idx = None
    sel_vals = []
    count = jnp.float32(32768.0)

    for t in range(_K):
        taken = taken_ref[...]
        if t == 0:
            parts = _colsum(x_ref, 4)
            colsum = parts[0]
            for p in parts[1:]:
                colsum = colsum + p
        elif t == 1:
            parts = _colsum(xz_ref, 8)
            for w in range(8):
                wsums_ref[w : w + 1, :] = parts[w]
            colsum = parts[0]
            for p in parts[1:]:
                colsum = colsum + p
        else:
            wstar = prev_idx // 4096
            acc = _window_chain(xz_ref, wstar * 4096, 4096)
            wsums_ref[pl.ds(wstar, 1), :] = _tree(acc)
            v = wsums_ref[...]
            colsum = v[0:1]
            for w in range(1, 8):
                colsum = colsum + v[w : w + 1]
        mean_row = colsum / count
        chn = _N // 4
        ms = []
        for c in range(4):
            sc = lax.dot_general(
                mean_row,
                x_ref[pl.ds(c * chn, chn), :],
                (((1,), (1,)), ((), ())),
                preferred_element_type=jnp.float32,
            )
            zc = 1.0 / sc + taken[0:1, c * chn : (c + 1) * chn]
            z_ref[c : c + 1, :] = zc
            ms.append(jnp.max(zc))
        m = jnp.maximum(jnp.maximum(ms[0], ms[1]), jnp.maximum(ms[2], ms[3]))
        wc = jnp.int32(3)
        for c in (2, 1, 0):
            wc = jnp.where(ms[c] == m, jnp.int32(c), wc)
        zrow = z_ref[pl.ds(wc, 1), :]
        cand = jnp.where(zrow == m, iota_chunk, jnp.full((1, chn), _INT_MAX, jnp.int32))
        idx = jnp.min(cand) + wc * chn
        sel_ref[t] = idx
        prev_idx = idx
        taken_ref[...] = jnp.where(iota_row == idx, taken - 100000.0, taken)
        is_dup = jnp.bool_(False)
        for v in sel_vals:
            is_dup = jnp.logical_or(is_dup, v == idx)
        count = count - jnp.where(is_dup, 0.0, 1.0).astype(jnp.float32)
        sel_vals.append(idx)
        if t == 0:
            copy.wait()
        xz_ref[pl.ds(idx, 1), :] = jnp.zeros((1, _D), jnp.float32)

    vals = [sel_ref[k] for k in range(_K)]
    for k in range(_K):
        rank = jnp.int32(0)
        for j in range(_K):
            if j < k:
                rank += (vals[j] <= vals[k]).astype(jnp.int32)
            elif j > k:
                rank += (vals[j] < vals[k]).astype(jnp.int32)
        out_ref[pl.ds(rank, 1), :] = x_ref[pl.ds(vals[k], 1), :]


@jax.jit
def kernel(x):
    return pl.pallas_call(
        _body,
        out_shape=jax.ShapeDtypeStruct((_K, _D), jnp.float32),
        in_specs=[pl.BlockSpec(memory_space=pltpu.VMEM)],
        out_specs=pl.BlockSpec(memory_space=pltpu.VMEM),
        scratch_shapes=[
            pltpu.VMEM((_N, _D), jnp.float32),
            pltpu.VMEM((1, _N), jnp.float32),
            pltpu.VMEM((4, _N // 4), jnp.float32),
            pltpu.SMEM((_K,), jnp.int32),
            pltpu.VMEM((8, _D), jnp.float32),
            pltpu.SemaphoreType.DMA,
        ],
    )(x)
